# baseline (speedup 1.0000x reference)
import functools

import jax
import jax.numpy as jnp
from jax.experimental import pallas as pl
from jax.experimental.pallas import tpu as pltpu


def _dot(a, b, dims, precision=None):
    return jax.lax.dot_general(a, b, (dims, ((), ())),
                               precision=precision,
                               preferred_element_type=jnp.float32)


_EXACT = jax.lax.Precision.HIGHEST


def _fused(n_layers, x_ref, ei_ref, ea_ref, *refs):
    p = refs[:6 * n_layers]
    ws_ref, bs_ref, out_ref = refs[6 * n_layers:6 * n_layers + 3]
    w_vmem = refs[6 * n_layers + 3:6 * n_layers + 3 + n_layers]
    sems = refs[-1]

    N = x_ref.shape[0]
    E = ei_ref.shape[1]

    copies = []
    for i in range(n_layers):
        cp = pltpu.make_async_copy(p[6 * i], w_vmem[i], sems.at[i])
        cp.start()
        copies.append(cp)

    src_row = ei_ref[0:1, :]
    dst_row = ei_ref[1:2, :]
    iota_n = jax.lax.broadcasted_iota(jnp.int32, (N, E), 0)
    S_T = (iota_n == src_row).astype(jnp.float32)
    D_b = iota_n == dst_row
    D_T = D_b.astype(jnp.float32)

    ea = ea_ref[:]
    h = x_ref[:]

    for i in range(n_layers):
        _, as_r, ad_r, we_r, ae_r, b_r = p[6 * i:6 * i + 6]
        As = as_r[:].reshape(1, -1)
        Ad = ad_r[:].reshape(1, -1)
        Ae = ae_r[:].reshape(1, -1)
        b = b_r[:].reshape(1, -1)

        ep = _dot(ea, we_r[:], ((1,), (0,)))
        a_edge = _dot(Ae, ep, ((1,), (1,)), _EXACT)

        copies[i].wait()
        W = w_vmem[i][:]
        hp = _dot(h, W, ((1,), (0,)))

        a_src = jnp.sum(hp * As, axis=1, keepdims=True)
        a_dst = jnp.sum(hp * Ad, axis=1, keepdims=True)

        alpha = (_dot(a_src, S_T, ((0,), (0,)), _EXACT)
                 + _dot(a_dst, D_T, ((0,), (0,)), _EXACT)
                 + a_edge)
        alpha = jnp.where(alpha >= 0, alpha, 0.2 * alpha)

        masked = jnp.where(D_b, alpha, -jnp.inf)
        m = jnp.max(masked, axis=1, keepdims=True)
        m = jnp.where(jnp.isfinite(m), m, 0.0)
        m_dst = _dot(m, D_T, ((0,), (0,)), _EXACT)
        ex = jnp.exp(alpha - m_dst)
        denom = _dot(D_T, ex, ((1,), (1,)), _EXACT)
        denom_dst = _dot(denom, D_T, ((0,), (0,)), _EXACT)
        coef = ex / (denom_dst + 1e-16)

        mix = _dot(D_T * coef, S_T, ((1,), (1,)), _EXACT)
        h = _dot(mix, hp, ((1,), (0,)), _EXACT) + b
        if i < n_layers - 1:
            h = jnp.where(h > 0, h, 0.0)

    conn = _dot(h, h, ((1,), (1,)))
    scores = _dot(conn, ws_ref[:], ((1,), (0,))) + bs_ref[:].reshape(1, 1)
    out_ref[:] = jax.nn.sigmoid(scores)


def kernel(x, edge_index, edge_attr, params, Ws, bs):
    N = x.shape[0]
    n_layers = len(params)
    flat = [q for layer in params for q in layer]

    vmem = pl.BlockSpec(memory_space=pltpu.MemorySpace.VMEM)
    hbm = pl.BlockSpec(memory_space=pltpu.MemorySpace.HBM)
    in_specs = [vmem, vmem, vmem]
    for _ in params:
        in_specs += [hbm] + [vmem] * 5
    in_specs += [vmem, vmem]

    scratch_shapes = [pltpu.VMEM(layer[0].shape, jnp.float32) for layer in params]
    scratch_shapes.append(pltpu.SemaphoreType.DMA((n_layers,)))

    out = pl.pallas_call(
        functools.partial(_fused, n_layers),
        out_shape=jax.ShapeDtypeStruct((N, 1), jnp.float32),
        in_specs=in_specs,
        scratch_shapes=scratch_shapes,
    )(x, edge_index, edge_attr, *flat, Ws, bs)
    return out


# speedup vs baseline: 1.0681x; 1.0681x over previous
import jax
import jax.numpy as jnp
from jax.experimental import pallas as pl


def _dot(a, b, dims, precision=None):
    return jax.lax.dot_general(a, b, (dims, ((), ())),
                               precision=precision,
                               preferred_element_type=jnp.float32)


_EXACT = jax.lax.Precision.HIGHEST


def _fused(x_ref, ei_ref, ea_ref, *refs):
    n_layers = (len(refs) - 3) // 6
    out_ref = refs[-1]
    ws_ref, bs_ref = refs[-3], refs[-2]

    N = x_ref.shape[0]
    E = ei_ref.shape[1]

    src_row = ei_ref[0:1, :]
    dst_row = ei_ref[1:2, :]
    iota_n = jax.lax.broadcasted_iota(jnp.int32, (N, E), 0)
    S_T = (iota_n == src_row).astype(jnp.float32)
    D_b = iota_n == dst_row
    D_T = D_b.astype(jnp.float32)
    ones_col = jnp.ones((E, 1), jnp.float32)

    ea = ea_ref[:]
    h = x_ref[:]

    for i in range(n_layers):
        w_r, as_r, ad_r, we_r, ae_r, b_r = refs[6 * i:6 * i + 6]
        W = w_r[:]
        As = as_r[:].reshape(1, -1)
        Ad = ad_r[:].reshape(1, -1)
        Ae = ae_r[:].reshape(1, -1)
        b = b_r[:].reshape(1, -1)

        hp = _dot(h, W, ((1,), (0,)))
        ep = _dot(ea, we_r[:], ((1,), (0,)))

        a_src = jnp.sum(hp * As, axis=1, keepdims=True)
        a_dst = jnp.sum(hp * Ad, axis=1, keepdims=True)
        a_edge = _dot(Ae, ep, ((1,), (1,)), _EXACT)

        alpha = (_dot(a_src, S_T, ((0,), (0,)), _EXACT)
                 + _dot(a_dst, D_T, ((0,), (0,)), _EXACT)
                 + a_edge)
        alpha = jnp.where(alpha >= 0, alpha, 0.2 * alpha)

        masked = jnp.where(D_b, alpha, -jnp.inf)
        m = jnp.max(masked, axis=1, keepdims=True)
        m = jnp.where(jnp.isfinite(m), m, 0.0)
        exm = jnp.exp(masked - m)
        denom = _dot(exm, ones_col, ((1,), (0,)), _EXACT)
        w_ne = exm / (denom + 1e-16)

        mix = _dot(w_ne, S_T, ((1,), (1,)), _EXACT)
        h = _dot(mix, hp, ((1,), (0,)), _EXACT) + b
        if i < n_layers - 1:
            h = jnp.where(h > 0, h, 0.0)

    conn = _dot(h, h, ((1,), (1,)))
    scores = _dot(conn, ws_ref[:], ((1,), (0,))) + bs_ref[:].reshape(1, 1)
    out_ref[:] = jax.nn.sigmoid(scores)


def kernel(x, edge_index, edge_attr, params, Ws, bs):
    N = x.shape[0]
    flat = [q for layer in params for q in layer]
    out = pl.pallas_call(
        _fused,
        out_shape=jax.ShapeDtypeStruct((N, 1), jnp.float32),
    )(x, edge_index, edge_attr, *flat, Ws, bs)
    return out
